# initial kernel scaffold (unmeasured)
import jax
import jax.numpy as jnp
from jax import lax
from jax.experimental import pallas as pl
from jax.experimental.pallas import tpu as pltpu


def kernel(
    x,
):
    def body(*refs):
        pass

    out_shape = jax.ShapeDtypeStruct(..., jnp.float32)
    return pl.pallas_call(body, out_shape=out_shape)(...)


# baseline (device time: 10430 ns/iter reference)
import jax
import jax.numpy as jnp
from jax import lax
from jax.experimental import pallas as pl
from jax.experimental.pallas import tpu as pltpu

N_DEV = 16


def kernel(x):
    m_per, n = x.shape

    def body(x_ref, out_ref, gather_ref, send_sems, recv_sems):
        my_i = lax.axis_index("i")

        xv = x_ref[:, :]
        vmax = jnp.max(xv, axis=0)
        row = lax.broadcasted_iota(jnp.int32, (m_per, n), 0)
        big = jnp.float32(2 * m_per * N_DEV)
        ibig = jnp.int32(2 * m_per * N_DEV)
        lidx = jnp.min(jnp.where(xv == vmax[None, :], row, ibig), axis=0)
        gidx = lidx.astype(jnp.float32) + my_i.astype(jnp.float32) * jnp.float32(
            m_per
        )
        gather_ref[0, 0:1, :] = vmax[None, :]
        gather_ref[0, 1:2, :] = gidx[None, :]

        barrier_sem = pltpu.get_barrier_semaphore()
        for d in range(1, N_DEV):
            peer = lax.rem(my_i + d, N_DEV)
            pl.semaphore_signal(
                barrier_sem,
                inc=1,
                device_id=(peer,),
                device_id_type=pl.DeviceIdType.MESH,
            )
        pl.semaphore_wait(barrier_sem, N_DEV - 1)

        rdmas = []
        for d in range(1, N_DEV):
            peer = lax.rem(my_i + d, N_DEV)
            rdma = pltpu.make_async_remote_copy(
                src_ref=gather_ref.at[0],
                dst_ref=gather_ref.at[d],
                send_sem=send_sems.at[d],
                recv_sem=recv_sems.at[d],
                device_id=(peer,),
                device_id_type=pl.DeviceIdType.MESH,
            )
            rdma.start()
            rdmas.append(rdma)
        for rdma in rdmas:
            rdma.wait()

        buf = gather_ref[:, :, :]
        vals = buf[:, 0, :]
        idxs = buf[:, 1, :]
        gmax = jnp.max(vals, axis=0)
        win = jnp.min(jnp.where(vals == gmax[None, :], idxs, big), axis=0)
        out_ref[0:1, :] = gmax[None, :]
        out_ref[1:2, :] = win[None, :]

    return pl.pallas_call(
        body,
        out_shape=jax.ShapeDtypeStruct((2, n), jnp.float32),
        in_specs=[pl.BlockSpec(memory_space=pltpu.VMEM)],
        out_specs=pl.BlockSpec(memory_space=pltpu.VMEM),
        scratch_shapes=[
            pltpu.VMEM((N_DEV, 2, n), jnp.float32),
            pltpu.SemaphoreType.DMA((N_DEV,)),
            pltpu.SemaphoreType.DMA((N_DEV,)),
        ],
        compiler_params=pltpu.CompilerParams(collective_id=0),
    )(x)


# device time: 9779 ns/iter; 1.0666x vs baseline; 1.0666x over previous
import jax
import jax.numpy as jnp
from jax import lax
from jax.experimental import pallas as pl
from jax.experimental.pallas import tpu as pltpu

N_DEV = 16


def kernel(x):
    m_per, n = x.shape

    def body(x_ref, out_ref, gather_ref, send_sems, recv_sems):
        my_i = lax.axis_index("i")

        barrier_sem = pltpu.get_barrier_semaphore()
        for d in range(1, N_DEV):
            peer = lax.rem(my_i + d, N_DEV)
            pl.semaphore_signal(
                barrier_sem,
                inc=1,
                device_id=(peer,),
                device_id_type=pl.DeviceIdType.MESH,
            )

        xv = x_ref[:, :]
        vmax = jnp.max(xv, axis=0)
        row = lax.broadcasted_iota(jnp.int32, (m_per, n), 0)
        big = jnp.float32(2 * m_per * N_DEV)
        ibig = jnp.int32(2 * m_per * N_DEV)
        lidx = jnp.min(jnp.where(xv == vmax[None, :], row, ibig), axis=0)
        gidx = lidx.astype(jnp.float32) + my_i.astype(jnp.float32) * jnp.float32(
            m_per
        )
        gather_ref[0, 0:1, :] = vmax[None, :]
        gather_ref[0, 1:2, :] = gidx[None, :]

        pl.semaphore_wait(barrier_sem, N_DEV - 1)

        rdmas = []
        for d in range(1, N_DEV):
            peer = lax.rem(my_i + d, N_DEV)
            rdma = pltpu.make_async_remote_copy(
                src_ref=gather_ref.at[0],
                dst_ref=gather_ref.at[d],
                send_sem=send_sems.at[d],
                recv_sem=recv_sems.at[d],
                device_id=(peer,),
                device_id_type=pl.DeviceIdType.MESH,
            )
            rdma.start()
            rdmas.append(rdma)
        for rdma in rdmas:
            rdma.wait_recv()

        buf = gather_ref[:, :, :]
        vals = buf[:, 0, :]
        idxs = buf[:, 1, :]
        gmax = jnp.max(vals, axis=0)
        win = jnp.min(jnp.where(vals == gmax[None, :], idxs, big), axis=0)
        out_ref[0:1, :] = gmax[None, :]
        out_ref[1:2, :] = win[None, :]

        for rdma in rdmas:
            rdma.wait_send()

    return pl.pallas_call(
        body,
        out_shape=jax.ShapeDtypeStruct((2, n), jnp.float32),
        in_specs=[pl.BlockSpec(memory_space=pltpu.VMEM)],
        out_specs=pl.BlockSpec(memory_space=pltpu.VMEM),
        scratch_shapes=[
            pltpu.VMEM((N_DEV, 2, n), jnp.float32),
            pltpu.SemaphoreType.DMA((N_DEV,)),
            pltpu.SemaphoreType.DMA((N_DEV,)),
        ],
        compiler_params=pltpu.CompilerParams(collective_id=0),
    )(x)
